# Initial kernel scaffold; baseline (speedup 1.0000x reference)
import jax
import jax.numpy as jnp
from jax.experimental import pallas as pl


def kernel(x, edge_index, batch, shape_emb, color_emb, W_node, b_node, W1_rel, W1_root, b1, W2_rel, W2_root, b2, W_cls, b_cls):
    raise NotImplementedError("write your pallas kernel here")


# speedup vs baseline: 7.4086x; 7.4086x over previous
import functools

import jax
import jax.numpy as jnp
from jax import lax
from jax.experimental import pallas as pl
from jax.experimental.pallas import tpu as pltpu
from jax.experimental.pallas import tpu_sc as plsc

N = 50000
E = 800000
G = 128
NC = 2
NS = 16
LANE = 128
E_PAD = 819200
CH_TOT = E_PAD // LANE
AGG_ROWS = 50048
DUMP = N
WB = 3128
WB_LAST = N - 15 * WB
ZCHT = AGG_ROWS // LANE
IB = 8

BLK = 400
NBLK = N // BLK


@functools.lru_cache(maxsize=None)
def _build_sc_scatter(ch_per_sub, core_stride):
    mesh = plsc.VectorSubcoreMesh(core_axis_name="c", subcore_axis_name="s")
    out_t = (jax.ShapeDtypeStruct((N, 32), jnp.float32),
             jax.ShapeDtypeStruct((N, 32), jnp.float32))
    n_groups = ch_per_sub // IB
    scratch = [
        pltpu.VMEM((2, IB * LANE), jnp.int32),
        pltpu.VMEM((2, IB * LANE), jnp.int32),
        pltpu.VMEM((LANE, 32), jnp.float32),
        pltpu.VMEM((LANE, 32), jnp.float32),
        pltpu.VMEM((LANE, 32), jnp.float32),
        pltpu.VMEM((LANE, 32), jnp.float32),
        pltpu.VMEM_SHARED((AGG_ROWS, 32), jnp.float32),
        pltpu.SemaphoreType.DMA,
        pltpu.SemaphoreType.DMA,
        pltpu.SemaphoreType.DMA,
        pltpu.SemaphoreType.DMA,
        pltpu.SemaphoreType.DMA,
    ]

    def body(ha, hb, src, dst, out0, out1, srcv, dstv, r0, r1, r2, r3, agg,
             sg0, sg1, sg2, sg3, si):
        c = lax.axis_index("c")
        s = lax.axis_index("s")
        rows = (r0, r1, r2, r3)
        gsems = (sg0, sg1, sg2, sg3)

        def zrow(i, carry):
            r0[i, pl.ds(0, 16)] = jnp.zeros((16,), jnp.float32)
            r0[i, pl.ds(16, 16)] = jnp.zeros((16,), jnp.float32)
            return carry
        lax.fori_loop(0, LANE, zrow, 0)

        zc = jnp.where(s < 7, 25, 24)
        zoff = 24 * s + jnp.minimum(s, 7)

        def zchunk(i, carry):
            pltpu.sync_copy(r0, agg.at[pl.ds((zoff + i) * LANE, LANE)])
            return carry
        lax.fori_loop(0, zc, zchunk, 0)
        plsc.subcore_barrier()

        e0 = (c * core_stride + s * ch_per_sub) * LANE

        def stage(g):
            off = e0 + g * IB * LANE
            pltpu.async_copy(src.at[pl.ds(off, IB * LANE)], srcv.at[g % 2], si)
            pltpu.async_copy(dst.at[pl.ds(off, IB * LANE)], dstv.at[g % 2], si)

        def stage_wait():
            cp = pltpu.make_async_copy(src.at[pl.ds(0, IB * LANE)],
                                       srcv.at[0], si)
            cp.wait()
            cp.wait()

        def gather(j, buf, sem):
            gb = (j // IB) % 2
            o = (j % IB) * LANE
            idx = srcv.at[gb, pl.ds(o, LANE)]

            @pl.when(c == 0)
            def _ga():
                pltpu.async_copy(ha.at[idx], buf, sem)

            @pl.when(c == 1)
            def _gb():
                pltpu.async_copy(hb.at[idx], buf, sem)

        def gwait(buf, sem):
            pltpu.make_async_copy(ha.at[srcv.at[0, pl.ds(0, LANE)]], buf,
                                  sem).wait()

        def scatter(j, buf):
            gb = (j // IB) % 2
            o = (j % IB) * LANE
            pltpu.sync_copy(buf, agg.at[dstv.at[gb, pl.ds(o, LANE)]], add=True)

        pltpu.sync_copy(src.at[pl.ds(e0, IB * LANE)], srcv.at[0])
        pltpu.sync_copy(dst.at[pl.ds(e0, IB * LANE)], dstv.at[0])
        stage(1)
        for b in range(4):
            gather(b, rows[b], gsems[b])

        def quad(q, carry):
            j0 = q * 4
            for b in range(4):
                j = j0 + b
                gwait(rows[b], gsems[b])
                scatter(j, rows[b])
                jn = j + 4
                if b == 0:
                    g = j // IB

                    @pl.when(jnp.logical_and(
                            j % IB == 0,
                            jnp.logical_and(j > 0, g + 1 < n_groups)))
                    def _st():
                        stage(g + 1)

                    @pl.when(jnp.logical_and(j % IB == IB - 4, jn < ch_per_sub))
                    def _sw():
                        stage_wait()

                @pl.when(jn < ch_per_sub)
                def _gn():
                    gather(jn, rows[b], gsems[b])
            return carry
        lax.fori_loop(0, ch_per_sub // 4, quad, 0)
        plsc.subcore_barrier()

        @pl.when(jnp.logical_and(c == 0, s < NS - 1))
        def _w0():
            pltpu.sync_copy(agg.at[pl.ds(s * WB, WB)], out0.at[pl.ds(s * WB, WB)])

        @pl.when(jnp.logical_and(c == 0, s == NS - 1))
        def _w0l():
            pltpu.sync_copy(agg.at[pl.ds(s * WB, WB_LAST)],
                            out0.at[pl.ds(s * WB, WB_LAST)])

        @pl.when(jnp.logical_and(c == 1, s < NS - 1))
        def _w1():
            pltpu.sync_copy(agg.at[pl.ds(s * WB, WB)], out1.at[pl.ds(s * WB, WB)])

        @pl.when(jnp.logical_and(c == 1, s == NS - 1))
        def _w1l():
            pltpu.sync_copy(agg.at[pl.ds(s * WB, WB_LAST)],
                            out1.at[pl.ds(s * WB, WB_LAST)])

    return pl.kernel(body, out_type=out_t, mesh=mesh, scratch_types=scratch,
                     compiler_params=pltpu.CompilerParams(use_tc_tiling_on_sc=False))


def _tc_embed(x0f, x1f, se, ce, wn, bn):
    def body(x0_ref, x1_ref, se_ref, ce_ref, wn_ref, bn_ref, out_ref):
        i16 = lax.broadcasted_iota(jnp.int32, (BLK, 16), 1).astype(jnp.float32)
        oh0 = (x0_ref[...] == i16).astype(jnp.float32)
        oh1 = (x1_ref[...] == i16).astype(jnp.float32)
        a0 = jnp.dot(se_ref[...], wn_ref[0:8, :], preferred_element_type=jnp.float32)
        a1 = jnp.dot(ce_ref[...], wn_ref[8:16, :], preferred_element_type=jnp.float32)
        h = (jnp.dot(oh0, a0, preferred_element_type=jnp.float32)
             + jnp.dot(oh1, a1, preferred_element_type=jnp.float32)
             + bn_ref[...])
        out_ref[...] = jnp.maximum(h, 0.0)

    return pl.pallas_call(
        body,
        grid=(NBLK,),
        in_specs=[
            pl.BlockSpec((BLK, 1), lambda i: (i, 0)),
            pl.BlockSpec((BLK, 1), lambda i: (i, 0)),
            pl.BlockSpec((16, 8), lambda i: (0, 0)),
            pl.BlockSpec((16, 8), lambda i: (0, 0)),
            pl.BlockSpec((16, 32), lambda i: (0, 0)),
            pl.BlockSpec((1, 32), lambda i: (0, 0)),
        ],
        out_specs=pl.BlockSpec((BLK, 32), lambda i: (i, 0)),
        out_shape=jax.ShapeDtypeStruct((N, 32), jnp.float32),
    )(x0f, x1f, se, ce, wn, bn)


def _tc_layer1(p0, p1, h0, w_rel, w_root, b):
    def body(p0_ref, p1_ref, h0_ref, wrel_ref, wroot_ref, b_ref, lo_ref, hi_ref):
        agg = p0_ref[...] + p1_ref[...]
        h = (jnp.dot(agg, wrel_ref[...], preferred_element_type=jnp.float32)
             + jnp.dot(h0_ref[...], wroot_ref[...], preferred_element_type=jnp.float32)
             + b_ref[...])
        h = jnp.maximum(h, 0.0)
        lo_ref[...] = h[:, 0:32]
        hi_ref[...] = h[:, 32:64]

    return pl.pallas_call(
        body,
        grid=(NBLK,),
        in_specs=[
            pl.BlockSpec((BLK, 32), lambda i: (i, 0)),
            pl.BlockSpec((BLK, 32), lambda i: (i, 0)),
            pl.BlockSpec((BLK, 32), lambda i: (i, 0)),
            pl.BlockSpec((32, 64), lambda i: (0, 0)),
            pl.BlockSpec((32, 64), lambda i: (0, 0)),
            pl.BlockSpec((1, 64), lambda i: (0, 0)),
        ],
        out_specs=[
            pl.BlockSpec((BLK, 32), lambda i: (i, 0)),
            pl.BlockSpec((BLK, 32), lambda i: (i, 0)),
        ],
        out_shape=[
            jax.ShapeDtypeStruct((N, 32), jnp.float32),
            jax.ShapeDtypeStruct((N, 32), jnp.float32),
        ],
    )(p0, p1, h0, w_rel, w_root, b)


def _tc_layer2_pool(q0, q1, h1lo, h1hi, batch3, w_rel, w_root, b2, wcls, bcls):
    def body(q0_ref, q1_ref, lo_ref, hi_ref, bt_ref, wrel_ref, wroot_ref,
             b2_ref, wcls_ref, bcls_ref, out_ref, sums, counts):
        i = pl.program_id(0)

        @pl.when(i == 0)
        def _init():
            sums[...] = jnp.zeros_like(sums)
            counts[...] = jnp.zeros_like(counts)

        h = (jnp.dot(q0_ref[...], wrel_ref[0:32, :], preferred_element_type=jnp.float32)
             + jnp.dot(q1_ref[...], wrel_ref[32:64, :], preferred_element_type=jnp.float32)
             + jnp.dot(lo_ref[...], wroot_ref[0:32, :], preferred_element_type=jnp.float32)
             + jnp.dot(hi_ref[...], wroot_ref[32:64, :], preferred_element_type=jnp.float32)
             + b2_ref[...])
        h = jnp.maximum(h, 0.0)

        brow = bt_ref[...].reshape(1, BLK)
        ig = lax.broadcasted_iota(jnp.int32, (G, BLK), 0).astype(jnp.float32)
        pt = (brow == ig).astype(jnp.float32)
        sums[...] += jnp.dot(pt, h, preferred_element_type=jnp.float32)
        counts[...] += jnp.sum(pt, axis=1, keepdims=True)

        @pl.when(i == NBLK - 1)
        def _fin():
            pooled = sums[...] / jnp.maximum(counts[...], 1.0)
            out_ref[...] = (jnp.dot(pooled, wcls_ref[...],
                                    preferred_element_type=jnp.float32)
                            + bcls_ref[...])

    return pl.pallas_call(
        body,
        grid=(NBLK,),
        in_specs=[
            pl.BlockSpec((BLK, 32), lambda i: (i, 0)),
            pl.BlockSpec((BLK, 32), lambda i: (i, 0)),
            pl.BlockSpec((BLK, 32), lambda i: (i, 0)),
            pl.BlockSpec((BLK, 32), lambda i: (i, 0)),
            pl.BlockSpec((1, 1, BLK), lambda i: (i, 0, 0)),
            pl.BlockSpec((64, 64), lambda i: (0, 0)),
            pl.BlockSpec((64, 64), lambda i: (0, 0)),
            pl.BlockSpec((1, 64), lambda i: (0, 0)),
            pl.BlockSpec((64, 10), lambda i: (0, 0)),
            pl.BlockSpec((1, 10), lambda i: (0, 0)),
        ],
        out_specs=pl.BlockSpec((G, 10), lambda i: (0, 0)),
        out_shape=jax.ShapeDtypeStruct((G, 10), jnp.float32),
        scratch_shapes=[
            pltpu.VMEM((G, 64), jnp.float32),
            pltpu.VMEM((G, 1), jnp.float32),
        ],
    )(q0, q1, h1lo, h1hi, batch3, w_rel, w_root, b2, wcls, bcls)


def kernel(x, edge_index, batch, shape_emb, color_emb, W_node, b_node,
           W1_rel, W1_root, b1, W2_rel, W2_root, b2, W_cls, b_cls):
    f32 = jnp.float32
    x0f = x[:, 0:1].astype(f32)
    x1f = x[:, 1:2].astype(f32)
    batch3 = batch.astype(f32).reshape(NBLK, 1, BLK)
    src = edge_index[0].astype(jnp.int32)
    dst = edge_index[1].astype(jnp.int32)
    pad = E_PAD - E
    srcp = jnp.concatenate([src, jnp.zeros((pad,), jnp.int32)])
    dstp = jnp.concatenate([dst, jnp.full((pad,), DUMP, jnp.int32)])
    bn = b_node.reshape(1, -1)
    b1r = b1.reshape(1, -1)
    b2r = b2.reshape(1, -1)
    bc = b_cls.reshape(1, -1)

    h0 = _tc_embed(x0f, x1f, shape_emb, color_emb, W_node, bn)
    p0, p1 = _build_sc_scatter(200, 3200)(h0, h0, srcp, dstp)
    h1lo, h1hi = _tc_layer1(p0, p1, h0, W1_rel, W1_root, b1r)
    q0, q1 = _build_sc_scatter(400, 0)(h1lo, h1hi, srcp, dstp)
    return _tc_layer2_pool(q0, q1, h1lo, h1hi, batch3, W2_rel, W2_root,
                           b2r, W_cls, bc)


# speedup vs baseline: 8.4189x; 1.1364x over previous
import functools

import jax
import jax.numpy as jnp
from jax import lax
from jax.experimental import pallas as pl
from jax.experimental.pallas import tpu as pltpu
from jax.experimental.pallas import tpu_sc as plsc

N = 50000
E = 800000
G = 128
NC = 2
NS = 16
LANE = 128
E_PAD = 819200
CH_TOT = E_PAD // LANE
AGG_ROWS = 50048
NDUMP = 48
WB = 3128
WB_LAST = N - 15 * WB
ZCHT = AGG_ROWS // LANE
IB = 8

BLK = 5000
NBLK = N // BLK


@functools.lru_cache(maxsize=None)
def _build_sc_scatter(ch_per_sub, core_stride):
    mesh = plsc.VectorSubcoreMesh(core_axis_name="c", subcore_axis_name="s")
    out_t = (jax.ShapeDtypeStruct((N, 32), jnp.float32),
             jax.ShapeDtypeStruct((N, 32), jnp.float32))
    n_groups = ch_per_sub // IB
    scratch = [
        pltpu.VMEM((2, IB * LANE), jnp.int32),
        pltpu.VMEM((2, IB * LANE), jnp.int32),
        pltpu.VMEM((LANE, 32), jnp.float32),
        pltpu.VMEM((LANE, 32), jnp.float32),
        pltpu.VMEM((LANE, 32), jnp.float32),
        pltpu.VMEM((LANE, 32), jnp.float32),
        pltpu.VMEM_SHARED((AGG_ROWS, 32), jnp.float32),
        pltpu.SemaphoreType.DMA,
        pltpu.SemaphoreType.DMA,
        pltpu.SemaphoreType.DMA,
        pltpu.SemaphoreType.DMA,
        pltpu.SemaphoreType.DMA,
    ]

    def body(ha, hb, src, dst, out0, out1, srcv, dstv, r0, r1, r2, r3, agg,
             sg0, sg1, sg2, sg3, si):
        c = lax.axis_index("c")
        s = lax.axis_index("s")
        rows = (r0, r1, r2, r3)
        gsems = (sg0, sg1, sg2, sg3)

        def zrow(i, carry):
            r0[i, pl.ds(0, 16)] = jnp.zeros((16,), jnp.float32)
            r0[i, pl.ds(16, 16)] = jnp.zeros((16,), jnp.float32)
            return carry
        lax.fori_loop(0, LANE, zrow, 0)

        zc = jnp.where(s < 7, 25, 24)
        zoff = 24 * s + jnp.minimum(s, 7)

        def zchunk(i, carry):
            pltpu.sync_copy(r0, agg.at[pl.ds((zoff + i) * LANE, LANE)])
            return carry
        lax.fori_loop(0, zc, zchunk, 0)
        plsc.subcore_barrier()

        e0 = (c * core_stride + s * ch_per_sub) * LANE

        def stage(g):
            off = e0 + g * IB * LANE
            pltpu.async_copy(src.at[pl.ds(off, IB * LANE)], srcv.at[g % 2], si)
            pltpu.async_copy(dst.at[pl.ds(off, IB * LANE)], dstv.at[g % 2], si)

        def stage_wait():
            cp = pltpu.make_async_copy(src.at[pl.ds(0, IB * LANE)],
                                       srcv.at[0], si)
            cp.wait()
            cp.wait()

        def gather(j, buf, sem):
            gb = (j // IB) % 2
            o = (j % IB) * LANE
            idx = srcv.at[gb, pl.ds(o, LANE)]

            @pl.when(c == 0)
            def _ga():
                pltpu.async_copy(ha.at[idx], buf, sem)

            @pl.when(c == 1)
            def _gb():
                pltpu.async_copy(hb.at[idx], buf, sem)

        def gwait(buf, sem):
            pltpu.make_async_copy(ha.at[srcv.at[0, pl.ds(0, LANE)]], buf,
                                  sem).wait()

        def scatter(j, buf):
            gb = (j // IB) % 2
            o = (j % IB) * LANE
            pltpu.sync_copy(buf, agg.at[dstv.at[gb, pl.ds(o, LANE)]], add=True)

        pltpu.sync_copy(src.at[pl.ds(e0, IB * LANE)], srcv.at[0])
        pltpu.sync_copy(dst.at[pl.ds(e0, IB * LANE)], dstv.at[0])
        stage(1)
        for b in range(4):
            gather(b, rows[b], gsems[b])

        def quad(q, carry):
            j0 = q * 4
            for b in range(4):
                j = j0 + b
                gwait(rows[b], gsems[b])
                scatter(j, rows[b])
                jn = j + 4
                if b == 0:
                    g = j // IB

                    @pl.when(jnp.logical_and(
                            j % IB == 0,
                            jnp.logical_and(j > 0, g + 1 < n_groups)))
                    def _st():
                        stage(g + 1)

                    @pl.when(jnp.logical_and(j % IB == IB - 4, jn < ch_per_sub))
                    def _sw():
                        stage_wait()

                @pl.when(jn < ch_per_sub)
                def _gn():
                    gather(jn, rows[b], gsems[b])
            return carry
        lax.fori_loop(0, ch_per_sub // 4, quad, 0)
        plsc.subcore_barrier()

        @pl.when(jnp.logical_and(c == 0, s < NS - 1))
        def _w0():
            pltpu.sync_copy(agg.at[pl.ds(s * WB, WB)], out0.at[pl.ds(s * WB, WB)])

        @pl.when(jnp.logical_and(c == 0, s == NS - 1))
        def _w0l():
            pltpu.sync_copy(agg.at[pl.ds(s * WB, WB_LAST)],
                            out0.at[pl.ds(s * WB, WB_LAST)])

        @pl.when(jnp.logical_and(c == 1, s < NS - 1))
        def _w1():
            pltpu.sync_copy(agg.at[pl.ds(s * WB, WB)], out1.at[pl.ds(s * WB, WB)])

        @pl.when(jnp.logical_and(c == 1, s == NS - 1))
        def _w1l():
            pltpu.sync_copy(agg.at[pl.ds(s * WB, WB_LAST)],
                            out1.at[pl.ds(s * WB, WB_LAST)])

    return pl.kernel(body, out_type=out_t, mesh=mesh, scratch_types=scratch,
                     compiler_params=pltpu.CompilerParams(use_tc_tiling_on_sc=False))


def _tc_embed(x, se, ce, wn, bn):
    def body(x_ref, se_ref, ce_ref, wn_ref, bn_ref, out_ref):
        i16 = lax.broadcasted_iota(jnp.int32, (BLK, 16), 1)
        oh0 = (x_ref[:, 0:1] == i16).astype(jnp.float32)
        oh1 = (x_ref[:, 1:2] == i16).astype(jnp.float32)
        a0 = jnp.dot(se_ref[...], wn_ref[0:8, :], preferred_element_type=jnp.float32)
        a1 = jnp.dot(ce_ref[...], wn_ref[8:16, :], preferred_element_type=jnp.float32)
        h = (jnp.dot(oh0, a0, preferred_element_type=jnp.float32)
             + jnp.dot(oh1, a1, preferred_element_type=jnp.float32)
             + bn_ref[...])
        out_ref[...] = jnp.maximum(h, 0.0)

    return pl.pallas_call(
        body,
        grid=(NBLK,),
        in_specs=[
            pl.BlockSpec((BLK, 2), lambda i: (i, 0)),
            pl.BlockSpec((16, 8), lambda i: (0, 0)),
            pl.BlockSpec((16, 8), lambda i: (0, 0)),
            pl.BlockSpec((16, 32), lambda i: (0, 0)),
            pl.BlockSpec((1, 32), lambda i: (0, 0)),
        ],
        out_specs=pl.BlockSpec((BLK, 32), lambda i: (i, 0)),
        out_shape=jax.ShapeDtypeStruct((N, 32), jnp.float32),
    )(x, se, ce, wn, bn)


def _tc_root(lo, hi, w_root, b):
    def body(lo_ref, hi_ref, wroot_ref, b_ref, out_ref):
        out_ref[...] = (
            jnp.dot(lo_ref[...], wroot_ref[0:32, :], preferred_element_type=jnp.float32)
            + jnp.dot(hi_ref[...], wroot_ref[32:64, :], preferred_element_type=jnp.float32)
            + b_ref[...])

    return pl.pallas_call(
        body,
        grid=(NBLK,),
        in_specs=[
            pl.BlockSpec((BLK, 32), lambda i: (i, 0)),
            pl.BlockSpec((BLK, 32), lambda i: (i, 0)),
            pl.BlockSpec((64, 64), lambda i: (0, 0)),
            pl.BlockSpec((1, 64), lambda i: (0, 0)),
        ],
        out_specs=pl.BlockSpec((BLK, 64), lambda i: (i, 0)),
        out_shape=jax.ShapeDtypeStruct((N, 64), jnp.float32),
    )(lo, hi, w_root, b)


def _tc_layer1(p0, p1, h0, w_rel, w_root, b):
    def body(p0_ref, p1_ref, h0_ref, wrel_ref, wroot_ref, b_ref, lo_ref, hi_ref):
        agg = p0_ref[...] + p1_ref[...]
        h = (jnp.dot(agg, wrel_ref[...], preferred_element_type=jnp.float32)
             + jnp.dot(h0_ref[...], wroot_ref[...], preferred_element_type=jnp.float32)
             + b_ref[...])
        h = jnp.maximum(h, 0.0)
        lo_ref[...] = h[:, 0:32]
        hi_ref[...] = h[:, 32:64]

    return pl.pallas_call(
        body,
        grid=(NBLK,),
        in_specs=[
            pl.BlockSpec((BLK, 32), lambda i: (i, 0)),
            pl.BlockSpec((BLK, 32), lambda i: (i, 0)),
            pl.BlockSpec((BLK, 32), lambda i: (i, 0)),
            pl.BlockSpec((32, 64), lambda i: (0, 0)),
            pl.BlockSpec((32, 64), lambda i: (0, 0)),
            pl.BlockSpec((1, 64), lambda i: (0, 0)),
        ],
        out_specs=[
            pl.BlockSpec((BLK, 32), lambda i: (i, 0)),
            pl.BlockSpec((BLK, 32), lambda i: (i, 0)),
        ],
        out_shape=[
            jax.ShapeDtypeStruct((N, 32), jnp.float32),
            jax.ShapeDtypeStruct((N, 32), jnp.float32),
        ],
    )(p0, p1, h0, w_rel, w_root, b)


def _tc_layer2_pool(q0, q1, r2, batch2, w_rel, wcls, bcls):
    def body(q0_ref, q1_ref, r2_ref, bt_ref, wrel_ref, wcls_ref, bcls_ref,
             out_ref, sums, counts):
        i = pl.program_id(0)

        @pl.when(i == 0)
        def _init():
            sums[...] = jnp.zeros_like(sums)
            counts[...] = jnp.zeros_like(counts)

        h = (jnp.dot(q0_ref[...], wrel_ref[0:32, :], preferred_element_type=jnp.float32)
             + jnp.dot(q1_ref[...], wrel_ref[32:64, :], preferred_element_type=jnp.float32)
             + r2_ref[...])
        h = jnp.maximum(h, 0.0)

        brow = bt_ref[...].reshape(1, BLK)
        ig = lax.broadcasted_iota(jnp.int32, (G, BLK), 0)
        pt = (brow == ig).astype(jnp.float32)
        sums[...] += jnp.dot(pt, h, preferred_element_type=jnp.float32)
        counts[...] += jnp.sum(pt, axis=1, keepdims=True)

        @pl.when(i == NBLK - 1)
        def _fin():
            pooled = sums[...] / jnp.maximum(counts[...], 1.0)
            out_ref[...] = (jnp.dot(pooled, wcls_ref[...],
                                    preferred_element_type=jnp.float32)
                            + bcls_ref[...])

    return pl.pallas_call(
        body,
        grid=(NBLK,),
        in_specs=[
            pl.BlockSpec((BLK, 32), lambda i: (i, 0)),
            pl.BlockSpec((BLK, 32), lambda i: (i, 0)),
            pl.BlockSpec((BLK, 64), lambda i: (i, 0)),
            pl.BlockSpec((1, 1, BLK), lambda i: (i, 0, 0)),
            pl.BlockSpec((64, 64), lambda i: (0, 0)),
            pl.BlockSpec((64, 10), lambda i: (0, 0)),
            pl.BlockSpec((1, 10), lambda i: (0, 0)),
        ],
        out_specs=pl.BlockSpec((G, 10), lambda i: (0, 0)),
        out_shape=jax.ShapeDtypeStruct((G, 10), jnp.float32),
        scratch_shapes=[
            pltpu.VMEM((G, 64), jnp.float32),
            pltpu.VMEM((G, 1), jnp.float32),
        ],
    )(q0, q1, r2, batch2, w_rel, wcls, bcls)


def kernel(x, edge_index, batch, shape_emb, color_emb, W_node, b_node,
           W1_rel, W1_root, b1, W2_rel, W2_root, b2, W_cls, b_cls):
    xi = x.astype(jnp.int32)
    batch2 = batch.astype(jnp.int32).reshape(NBLK, 1, BLK)
    src = edge_index[0].astype(jnp.int32)
    dst = edge_index[1].astype(jnp.int32)
    pad = E_PAD - E
    srcp = jnp.concatenate([src, jnp.zeros((pad,), jnp.int32)])
    dstp = jnp.concatenate(
        [dst, N + (jnp.arange(pad, dtype=jnp.int32) % NDUMP)])
    bn = b_node.reshape(1, -1)
    b1r = b1.reshape(1, -1)
    b2r = b2.reshape(1, -1)
    bc = b_cls.reshape(1, -1)

    h0 = _tc_embed(xi, shape_emb, color_emb, W_node, bn)
    p0, p1 = _build_sc_scatter(200, 3200)(h0, h0, srcp, dstp)
    h1lo, h1hi = _tc_layer1(p0, p1, h0, W1_rel, W1_root, b1r)
    r2 = _tc_root(h1lo, h1hi, W2_root, b2r)
    q0, q1 = _build_sc_scatter(400, 0)(h1lo, h1hi, srcp, dstp)
    return _tc_layer2_pool(q0, q1, r2, batch2, W2_rel, W_cls, bc)


# speedup vs baseline: 8.4583x; 1.0047x over previous
import functools

import jax
import jax.numpy as jnp
from jax import lax
from jax.experimental import pallas as pl
from jax.experimental.pallas import tpu as pltpu
from jax.experimental.pallas import tpu_sc as plsc

N = 50000
E = 800000
G = 128
NC = 2
NS = 16
LANE = 128
E_PAD = 819200
CH_TOT = E_PAD // LANE
AGG_ROWS = 50048
NDUMP = 48
WB = 3128
WB_LAST = N - 15 * WB
ZCHT = AGG_ROWS // LANE
IB = 8

BLK = 5000
NBLK = N // BLK


@functools.lru_cache(maxsize=None)
def _build_sc_scatter(ch_per_sub, core_stride):
    mesh = plsc.VectorSubcoreMesh(core_axis_name="c", subcore_axis_name="s")
    out_t = (jax.ShapeDtypeStruct((N, 32), jnp.float32),
             jax.ShapeDtypeStruct((N, 32), jnp.float32))
    n_groups = ch_per_sub // IB
    scratch = [
        pltpu.VMEM((2, IB * LANE), jnp.int32),
        pltpu.VMEM((2, IB * LANE), jnp.int32),
        pltpu.VMEM((LANE, 32), jnp.float32),
        pltpu.VMEM((LANE, 32), jnp.float32),
        pltpu.VMEM((LANE, 32), jnp.float32),
        pltpu.VMEM((LANE, 32), jnp.float32),
        pltpu.VMEM_SHARED((AGG_ROWS, 32), jnp.float32),
        pltpu.SemaphoreType.DMA,
        pltpu.SemaphoreType.DMA,
        pltpu.SemaphoreType.DMA,
        pltpu.SemaphoreType.DMA,
        pltpu.SemaphoreType.DMA,
    ]

    def body(ha, hb, src, dst, out0, out1, srcv, dstv, r0, r1, r2, r3, agg,
             sg0, sg1, sg2, sg3, si):
        c = lax.axis_index("c")
        s = lax.axis_index("s")
        rows = (r0, r1, r2, r3)
        gsems = (sg0, sg1, sg2, sg3)

        def zrow(i, carry):
            r0[i, pl.ds(0, 16)] = jnp.zeros((16,), jnp.float32)
            r0[i, pl.ds(16, 16)] = jnp.zeros((16,), jnp.float32)
            return carry
        lax.fori_loop(0, LANE, zrow, 0)

        zc = jnp.where(s < 7, 25, 24)
        zoff = 24 * s + jnp.minimum(s, 7)
        e0 = (c * core_stride + s * ch_per_sub) * LANE

        pltpu.async_copy(src.at[pl.ds(e0, IB * LANE)], srcv.at[0], sg0)
        pltpu.async_copy(dst.at[pl.ds(e0, IB * LANE)], dstv.at[0], sg0)

        def zchunk(i, carry):
            pltpu.async_copy(r0, agg.at[pl.ds((zoff + i) * LANE, LANE)], si)
            return carry
        lax.fori_loop(0, zc, zchunk, 0)

        def zwait(i, carry):
            pltpu.make_async_copy(r0, agg.at[pl.ds(zoff * LANE, LANE)],
                                  si).wait()
            return carry
        lax.fori_loop(0, zc, zwait, 0)

        cp0 = pltpu.make_async_copy(src.at[pl.ds(e0, IB * LANE)],
                                    srcv.at[0], sg0)
        cp0.wait()
        cp0.wait()

        def stage(g):
            off = e0 + g * IB * LANE
            pltpu.async_copy(src.at[pl.ds(off, IB * LANE)], srcv.at[g % 2], si)
            pltpu.async_copy(dst.at[pl.ds(off, IB * LANE)], dstv.at[g % 2], si)

        def stage_wait():
            cp = pltpu.make_async_copy(src.at[pl.ds(0, IB * LANE)],
                                       srcv.at[0], si)
            cp.wait()
            cp.wait()

        def gather(j, buf, sem):
            gb = (j // IB) % 2
            o = (j % IB) * LANE
            idx = srcv.at[gb, pl.ds(o, LANE)]

            @pl.when(c == 0)
            def _ga():
                pltpu.async_copy(ha.at[idx], buf, sem)

            @pl.when(c == 1)
            def _gb():
                pltpu.async_copy(hb.at[idx], buf, sem)

        def gwait(buf, sem):
            pltpu.make_async_copy(ha.at[srcv.at[0, pl.ds(0, LANE)]], buf,
                                  sem).wait()

        def scatter(j, buf):
            gb = (j // IB) % 2
            o = (j % IB) * LANE
            pltpu.sync_copy(buf, agg.at[dstv.at[gb, pl.ds(o, LANE)]], add=True)

        stage(1)
        for b in range(4):
            gather(b, rows[b], gsems[b])
        plsc.subcore_barrier()

        def quad(q, carry):
            j0 = q * 4
            for b in range(4):
                j = j0 + b
                gwait(rows[b], gsems[b])
                scatter(j, rows[b])
                jn = j + 4
                if b == 0:
                    g = j // IB

                    @pl.when(jnp.logical_and(
                            j % IB == 0,
                            jnp.logical_and(j > 0, g + 1 < n_groups)))
                    def _st():
                        stage(g + 1)

                    @pl.when(jnp.logical_and(j % IB == IB - 4, jn < ch_per_sub))
                    def _sw():
                        stage_wait()

                @pl.when(jn < ch_per_sub)
                def _gn():
                    gather(jn, rows[b], gsems[b])
            return carry
        lax.fori_loop(0, ch_per_sub // 4, quad, 0)
        plsc.subcore_barrier()

        @pl.when(jnp.logical_and(c == 0, s < NS - 1))
        def _w0():
            pltpu.sync_copy(agg.at[pl.ds(s * WB, WB)], out0.at[pl.ds(s * WB, WB)])

        @pl.when(jnp.logical_and(c == 0, s == NS - 1))
        def _w0l():
            pltpu.sync_copy(agg.at[pl.ds(s * WB, WB_LAST)],
                            out0.at[pl.ds(s * WB, WB_LAST)])

        @pl.when(jnp.logical_and(c == 1, s < NS - 1))
        def _w1():
            pltpu.sync_copy(agg.at[pl.ds(s * WB, WB)], out1.at[pl.ds(s * WB, WB)])

        @pl.when(jnp.logical_and(c == 1, s == NS - 1))
        def _w1l():
            pltpu.sync_copy(agg.at[pl.ds(s * WB, WB_LAST)],
                            out1.at[pl.ds(s * WB, WB_LAST)])

    return pl.kernel(body, out_type=out_t, mesh=mesh, scratch_types=scratch,
                     compiler_params=pltpu.CompilerParams(use_tc_tiling_on_sc=False))


def _tc_embed(x, se, ce, wn, bn):
    def body(x_ref, se_ref, ce_ref, wn_ref, bn_ref, out_ref):
        i16 = lax.broadcasted_iota(jnp.int32, (BLK, 16), 1)
        oh0 = (x_ref[:, 0:1] == i16).astype(jnp.float32)
        oh1 = (x_ref[:, 1:2] == i16).astype(jnp.float32)
        a0 = jnp.dot(se_ref[...], wn_ref[0:8, :], preferred_element_type=jnp.float32)
        a1 = jnp.dot(ce_ref[...], wn_ref[8:16, :], preferred_element_type=jnp.float32)
        h = (jnp.dot(oh0, a0, preferred_element_type=jnp.float32)
             + jnp.dot(oh1, a1, preferred_element_type=jnp.float32)
             + bn_ref[...])
        out_ref[...] = jnp.maximum(h, 0.0)

    return pl.pallas_call(
        body,
        grid=(NBLK,),
        in_specs=[
            pl.BlockSpec((BLK, 2), lambda i: (i, 0)),
            pl.BlockSpec((16, 8), lambda i: (0, 0)),
            pl.BlockSpec((16, 8), lambda i: (0, 0)),
            pl.BlockSpec((16, 32), lambda i: (0, 0)),
            pl.BlockSpec((1, 32), lambda i: (0, 0)),
        ],
        out_specs=pl.BlockSpec((BLK, 32), lambda i: (i, 0)),
        out_shape=jax.ShapeDtypeStruct((N, 32), jnp.float32),
    )(x, se, ce, wn, bn)


def _tc_root(lo, hi, w_root, b):
    def body(lo_ref, hi_ref, wroot_ref, b_ref, out_ref):
        out_ref[...] = (
            jnp.dot(lo_ref[...], wroot_ref[0:32, :], preferred_element_type=jnp.float32)
            + jnp.dot(hi_ref[...], wroot_ref[32:64, :], preferred_element_type=jnp.float32)
            + b_ref[...])

    return pl.pallas_call(
        body,
        grid=(NBLK,),
        in_specs=[
            pl.BlockSpec((BLK, 32), lambda i: (i, 0)),
            pl.BlockSpec((BLK, 32), lambda i: (i, 0)),
            pl.BlockSpec((64, 64), lambda i: (0, 0)),
            pl.BlockSpec((1, 64), lambda i: (0, 0)),
        ],
        out_specs=pl.BlockSpec((BLK, 64), lambda i: (i, 0)),
        out_shape=jax.ShapeDtypeStruct((N, 64), jnp.float32),
    )(lo, hi, w_root, b)


def _tc_layer1(p0, p1, h0, w_rel, w_root, b):
    def body(p0_ref, p1_ref, h0_ref, wrel_ref, wroot_ref, b_ref, lo_ref, hi_ref):
        agg = p0_ref[...] + p1_ref[...]
        h = (jnp.dot(agg, wrel_ref[...], preferred_element_type=jnp.float32)
             + jnp.dot(h0_ref[...], wroot_ref[...], preferred_element_type=jnp.float32)
             + b_ref[...])
        h = jnp.maximum(h, 0.0)
        lo_ref[...] = h[:, 0:32]
        hi_ref[...] = h[:, 32:64]

    return pl.pallas_call(
        body,
        grid=(NBLK,),
        in_specs=[
            pl.BlockSpec((BLK, 32), lambda i: (i, 0)),
            pl.BlockSpec((BLK, 32), lambda i: (i, 0)),
            pl.BlockSpec((BLK, 32), lambda i: (i, 0)),
            pl.BlockSpec((32, 64), lambda i: (0, 0)),
            pl.BlockSpec((32, 64), lambda i: (0, 0)),
            pl.BlockSpec((1, 64), lambda i: (0, 0)),
        ],
        out_specs=[
            pl.BlockSpec((BLK, 32), lambda i: (i, 0)),
            pl.BlockSpec((BLK, 32), lambda i: (i, 0)),
        ],
        out_shape=[
            jax.ShapeDtypeStruct((N, 32), jnp.float32),
            jax.ShapeDtypeStruct((N, 32), jnp.float32),
        ],
    )(p0, p1, h0, w_rel, w_root, b)


def _tc_layer2_pool(q0, q1, r2, batch2, w_rel, wcls, bcls):
    def body(q0_ref, q1_ref, r2_ref, bt_ref, wrel_ref, wcls_ref, bcls_ref,
             out_ref, sums, counts):
        i = pl.program_id(0)

        @pl.when(i == 0)
        def _init():
            sums[...] = jnp.zeros_like(sums)
            counts[...] = jnp.zeros_like(counts)

        h = (jnp.dot(q0_ref[...], wrel_ref[0:32, :], preferred_element_type=jnp.float32)
             + jnp.dot(q1_ref[...], wrel_ref[32:64, :], preferred_element_type=jnp.float32)
             + r2_ref[...])
        h = jnp.maximum(h, 0.0)

        brow = bt_ref[...].reshape(1, BLK)
        ig = lax.broadcasted_iota(jnp.int32, (G, BLK), 0)
        pt = (brow == ig).astype(jnp.float32)
        sums[...] += jnp.dot(pt, h, preferred_element_type=jnp.float32)
        counts[...] += jnp.sum(pt, axis=1, keepdims=True)

        @pl.when(i == NBLK - 1)
        def _fin():
            pooled = sums[...] / jnp.maximum(counts[...], 1.0)
            out_ref[...] = (jnp.dot(pooled, wcls_ref[...],
                                    preferred_element_type=jnp.float32)
                            + bcls_ref[...])

    return pl.pallas_call(
        body,
        grid=(NBLK,),
        in_specs=[
            pl.BlockSpec((BLK, 32), lambda i: (i, 0)),
            pl.BlockSpec((BLK, 32), lambda i: (i, 0)),
            pl.BlockSpec((BLK, 64), lambda i: (i, 0)),
            pl.BlockSpec((1, 1, BLK), lambda i: (i, 0, 0)),
            pl.BlockSpec((64, 64), lambda i: (0, 0)),
            pl.BlockSpec((64, 10), lambda i: (0, 0)),
            pl.BlockSpec((1, 10), lambda i: (0, 0)),
        ],
        out_specs=pl.BlockSpec((G, 10), lambda i: (0, 0)),
        out_shape=jax.ShapeDtypeStruct((G, 10), jnp.float32),
        scratch_shapes=[
            pltpu.VMEM((G, 64), jnp.float32),
            pltpu.VMEM((G, 1), jnp.float32),
        ],
    )(q0, q1, r2, batch2, w_rel, wcls, bcls)


def kernel(x, edge_index, batch, shape_emb, color_emb, W_node, b_node,
           W1_rel, W1_root, b1, W2_rel, W2_root, b2, W_cls, b_cls):
    xi = x.astype(jnp.int32)
    batch2 = batch.astype(jnp.int32).reshape(NBLK, 1, BLK)
    src = edge_index[0].astype(jnp.int32)
    dst = edge_index[1].astype(jnp.int32)
    pad = E_PAD - E
    srcp = jnp.concatenate([src, jnp.zeros((pad,), jnp.int32)])
    dstp = jnp.concatenate(
        [dst, N + (jnp.arange(pad, dtype=jnp.int32) % NDUMP)])
    bn = b_node.reshape(1, -1)
    b1r = b1.reshape(1, -1)
    b2r = b2.reshape(1, -1)
    bc = b_cls.reshape(1, -1)

    h0 = _tc_embed(xi, shape_emb, color_emb, W_node, bn)
    p0, p1 = _build_sc_scatter(200, 3200)(h0, h0, srcp, dstp)
    h1lo, h1hi = _tc_layer1(p0, p1, h0, W1_rel, W1_root, b1r)
    r2 = _tc_root(h1lo, h1hi, W2_root, b2r)
    q0, q1 = _build_sc_scatter(400, 0)(h1lo, h1hi, srcp, dstp)
    return _tc_layer2_pool(q0, q1, r2, batch2, W2_rel, W_cls, bc)


# speedup vs baseline: 9.5028x; 1.1235x over previous
import functools

import jax
import jax.numpy as jnp
from jax import lax
from jax.experimental import pallas as pl
from jax.experimental.pallas import tpu as pltpu
from jax.experimental.pallas import tpu_sc as plsc

N = 50000
E = 800000
G = 128
NC = 2
NS = 16
LANE = 128
E_PAD = 819200
CH_TOT = E_PAD // LANE
AGG_ROWS = 50048
NDUMP = 48
WB = 3128
WB_LAST = N - 15 * WB
ZCHT = AGG_ROWS // LANE
IB = 8

BLK = 5000
NBLK = N // BLK


@functools.lru_cache(maxsize=None)
def _build_sc_scatter(ch_per_sub, core_stride):
    mesh = plsc.VectorSubcoreMesh(core_axis_name="c", subcore_axis_name="s")
    out_t = (jax.ShapeDtypeStruct((N, 32), jnp.float32),
             jax.ShapeDtypeStruct((N, 32), jnp.float32))
    n_groups = ch_per_sub // IB
    scratch = [
        pltpu.VMEM((2, IB * LANE), jnp.int32),
        pltpu.VMEM((2, IB * LANE), jnp.int32),
        pltpu.VMEM((LANE, 32), jnp.float32),
        pltpu.VMEM((LANE, 32), jnp.float32),
        pltpu.VMEM((LANE, 32), jnp.float32),
        pltpu.VMEM((LANE, 32), jnp.float32),
        pltpu.VMEM_SHARED((AGG_ROWS, 32), jnp.float32),
        pltpu.SemaphoreType.DMA,
        pltpu.SemaphoreType.DMA,
        pltpu.SemaphoreType.DMA,
        pltpu.SemaphoreType.DMA,
        pltpu.SemaphoreType.DMA,
    ]

    def body(ha, hb, src, dst, out0, out1, srcv, dstv, r0, r1, r2, r3, agg,
             sg0, sg1, sg2, sg3, si):
        c = lax.axis_index("c")
        s = lax.axis_index("s")
        rows = (r0, r1, r2, r3)
        gsems = (sg0, sg1, sg2, sg3)

        def zrow(i, carry):
            r0[i, pl.ds(0, 16)] = jnp.zeros((16,), jnp.float32)
            r0[i, pl.ds(16, 16)] = jnp.zeros((16,), jnp.float32)
            return carry
        lax.fori_loop(0, LANE, zrow, 0)

        zc = jnp.where(s < 7, 25, 24)
        zoff = 24 * s + jnp.minimum(s, 7)
        e0 = (c * core_stride + s * ch_per_sub) * LANE

        pltpu.async_copy(src.at[pl.ds(e0, IB * LANE)], srcv.at[0], sg0)
        pltpu.async_copy(dst.at[pl.ds(e0, IB * LANE)], dstv.at[0], sg0)

        def zchunk(i, carry):
            pltpu.async_copy(r0, agg.at[pl.ds((zoff + i) * LANE, LANE)], si)
            return carry
        lax.fori_loop(0, zc, zchunk, 0)

        def zwait(i, carry):
            pltpu.make_async_copy(r0, agg.at[pl.ds(zoff * LANE, LANE)],
                                  si).wait()
            return carry
        lax.fori_loop(0, zc, zwait, 0)

        cp0 = pltpu.make_async_copy(src.at[pl.ds(e0, IB * LANE)],
                                    srcv.at[0], sg0)
        cp0.wait()
        cp0.wait()

        def stage(g):
            off = e0 + g * IB * LANE
            pltpu.async_copy(src.at[pl.ds(off, IB * LANE)], srcv.at[g % 2], si)
            pltpu.async_copy(dst.at[pl.ds(off, IB * LANE)], dstv.at[g % 2], si)

        def stage_wait():
            cp = pltpu.make_async_copy(src.at[pl.ds(0, IB * LANE)],
                                       srcv.at[0], si)
            cp.wait()
            cp.wait()

        def gather(j, buf, sem):
            gb = (j // IB) % 2
            o = (j % IB) * LANE
            idx = srcv.at[gb, pl.ds(o, LANE)]

            @pl.when(c == 0)
            def _ga():
                pltpu.async_copy(ha.at[idx], buf, sem)

            @pl.when(c == 1)
            def _gb():
                pltpu.async_copy(hb.at[idx], buf, sem)

        def gwait(buf, sem):
            pltpu.make_async_copy(ha.at[srcv.at[0, pl.ds(0, LANE)]], buf,
                                  sem).wait()

        def scatter(j, buf):
            gb = (j // IB) % 2
            o = (j % IB) * LANE
            pltpu.sync_copy(buf, agg.at[dstv.at[gb, pl.ds(o, LANE)]], add=True)

        stage(1)
        for b in range(4):
            gather(b, rows[b], gsems[b])
        plsc.subcore_barrier()

        def quad(q, carry):
            j0 = q * 4
            for b in range(4):
                j = j0 + b
                gwait(rows[b], gsems[b])
                scatter(j, rows[b])
                jn = j + 4
                if b == 0:
                    g = j // IB

                    @pl.when(jnp.logical_and(
                            j % IB == 0,
                            jnp.logical_and(j > 0, g + 1 < n_groups)))
                    def _st():
                        stage(g + 1)

                    @pl.when(jnp.logical_and(j % IB == IB - 4, jn < ch_per_sub))
                    def _sw():
                        stage_wait()

                @pl.when(jn < ch_per_sub)
                def _gn():
                    gather(jn, rows[b], gsems[b])
            return carry
        lax.fori_loop(0, ch_per_sub // 4, quad, 0)
        plsc.subcore_barrier()

        @pl.when(jnp.logical_and(c == 0, s < NS - 1))
        def _w0():
            pltpu.sync_copy(agg.at[pl.ds(s * WB, WB)], out0.at[pl.ds(s * WB, WB)])

        @pl.when(jnp.logical_and(c == 0, s == NS - 1))
        def _w0l():
            pltpu.sync_copy(agg.at[pl.ds(s * WB, WB_LAST)],
                            out0.at[pl.ds(s * WB, WB_LAST)])

        @pl.when(jnp.logical_and(c == 1, s < NS - 1))
        def _w1():
            pltpu.sync_copy(agg.at[pl.ds(s * WB, WB)], out1.at[pl.ds(s * WB, WB)])

        @pl.when(jnp.logical_and(c == 1, s == NS - 1))
        def _w1l():
            pltpu.sync_copy(agg.at[pl.ds(s * WB, WB_LAST)],
                            out1.at[pl.ds(s * WB, WB_LAST)])

    return pl.kernel(body, out_type=out_t, mesh=mesh, scratch_types=scratch,
                     compiler_params=pltpu.CompilerParams(use_tc_tiling_on_sc=False))


def _tc_embed(x, se, ce, wn, bn):
    def body(x_ref, se_ref, ce_ref, wn_ref, bn_ref, out_ref):
        i16 = lax.broadcasted_iota(jnp.int32, (BLK, 16), 1)
        oh0 = (x_ref[:, 0:1] == i16).astype(jnp.float32)
        oh1 = (x_ref[:, 1:2] == i16).astype(jnp.float32)
        a0 = jnp.dot(se_ref[...], wn_ref[0:8, :], preferred_element_type=jnp.float32)
        a1 = jnp.dot(ce_ref[...], wn_ref[8:16, :], preferred_element_type=jnp.float32)
        h = (jnp.dot(oh0, a0, preferred_element_type=jnp.float32)
             + jnp.dot(oh1, a1, preferred_element_type=jnp.float32)
             + bn_ref[...])
        out_ref[...] = jnp.maximum(h, 0.0)

    return pl.pallas_call(
        body,
        grid=(NBLK,),
        in_specs=[
            pl.BlockSpec((BLK, 2), lambda i: (i, 0)),
            pl.BlockSpec((16, 8), lambda i: (0, 0)),
            pl.BlockSpec((16, 8), lambda i: (0, 0)),
            pl.BlockSpec((16, 32), lambda i: (0, 0)),
            pl.BlockSpec((1, 32), lambda i: (0, 0)),
        ],
        out_specs=pl.BlockSpec((BLK, 32), lambda i: (i, 0)),
        out_shape=jax.ShapeDtypeStruct((N, 32), jnp.float32),
    )(x, se, ce, wn, bn)


def _tc_root(lo, hi, w_root, b):
    def body(lo_ref, hi_ref, wroot_ref, b_ref, out_ref):
        out_ref[...] = (
            jnp.dot(lo_ref[...], wroot_ref[0:32, :], preferred_element_type=jnp.float32)
            + jnp.dot(hi_ref[...], wroot_ref[32:64, :], preferred_element_type=jnp.float32)
            + b_ref[...])

    return pl.pallas_call(
        body,
        grid=(NBLK,),
        in_specs=[
            pl.BlockSpec((BLK, 32), lambda i: (i, 0)),
            pl.BlockSpec((BLK, 32), lambda i: (i, 0)),
            pl.BlockSpec((64, 64), lambda i: (0, 0)),
            pl.BlockSpec((1, 64), lambda i: (0, 0)),
        ],
        out_specs=pl.BlockSpec((BLK, 64), lambda i: (i, 0)),
        out_shape=jax.ShapeDtypeStruct((N, 64), jnp.float32),
    )(lo, hi, w_root, b)


def _tc_layer1(p0, p1, h0, w_rel, w_root, b):
    def body(p0_ref, p1_ref, h0_ref, wrel_ref, wroot_ref, b_ref, lo_ref, hi_ref):
        agg = p0_ref[...] + p1_ref[...]
        h = (jnp.dot(agg, wrel_ref[...], preferred_element_type=jnp.float32)
             + jnp.dot(h0_ref[...], wroot_ref[...], preferred_element_type=jnp.float32)
             + b_ref[...])
        h = jnp.maximum(h, 0.0)
        lo_ref[...] = h[:, 0:32]
        hi_ref[...] = h[:, 32:64]

    return pl.pallas_call(
        body,
        grid=(NBLK,),
        in_specs=[
            pl.BlockSpec((BLK, 32), lambda i: (i, 0)),
            pl.BlockSpec((BLK, 32), lambda i: (i, 0)),
            pl.BlockSpec((BLK, 32), lambda i: (i, 0)),
            pl.BlockSpec((32, 64), lambda i: (0, 0)),
            pl.BlockSpec((32, 64), lambda i: (0, 0)),
            pl.BlockSpec((1, 64), lambda i: (0, 0)),
        ],
        out_specs=[
            pl.BlockSpec((BLK, 32), lambda i: (i, 0)),
            pl.BlockSpec((BLK, 32), lambda i: (i, 0)),
        ],
        out_shape=[
            jax.ShapeDtypeStruct((N, 32), jnp.float32),
            jax.ShapeDtypeStruct((N, 32), jnp.float32),
        ],
    )(p0, p1, h0, w_rel, w_root, b)


def _tc_layer2_pool(q0, q1, r2, batch2, w_rel, wcls, bcls):
    def body(q0_ref, q1_ref, r2_ref, bt_ref, wrel_ref, wcls_ref, bcls_ref,
             out_ref, sums, counts):
        i = pl.program_id(0)

        @pl.when(i == 0)
        def _init():
            sums[...] = jnp.zeros_like(sums)
            counts[...] = jnp.zeros_like(counts)

        h = (jnp.dot(q0_ref[...], wrel_ref[0:32, :], preferred_element_type=jnp.float32)
             + jnp.dot(q1_ref[...], wrel_ref[32:64, :], preferred_element_type=jnp.float32)
             + r2_ref[...])
        h = jnp.maximum(h, 0.0)

        brow = bt_ref[...].reshape(1, BLK)
        ig = lax.broadcasted_iota(jnp.int32, (G, BLK), 0)
        pt = (brow == ig).astype(jnp.float32)
        sums[...] += jnp.dot(pt, h, preferred_element_type=jnp.float32)
        counts[...] += jnp.sum(pt, axis=1, keepdims=True)

        @pl.when(i == NBLK - 1)
        def _fin():
            pooled = sums[...] / jnp.maximum(counts[...], 1.0)
            out_ref[...] = (jnp.dot(pooled, wcls_ref[...],
                                    preferred_element_type=jnp.float32)
                            + bcls_ref[...])

    return pl.pallas_call(
        body,
        grid=(NBLK,),
        in_specs=[
            pl.BlockSpec((BLK, 32), lambda i: (i, 0)),
            pl.BlockSpec((BLK, 32), lambda i: (i, 0)),
            pl.BlockSpec((BLK, 64), lambda i: (i, 0)),
            pl.BlockSpec((1, 1, BLK), lambda i: (i, 0, 0)),
            pl.BlockSpec((64, 64), lambda i: (0, 0)),
            pl.BlockSpec((64, 10), lambda i: (0, 0)),
            pl.BlockSpec((1, 10), lambda i: (0, 0)),
        ],
        out_specs=pl.BlockSpec((G, 10), lambda i: (0, 0)),
        out_shape=jax.ShapeDtypeStruct((G, 10), jnp.float32),
        scratch_shapes=[
            pltpu.VMEM((G, 64), jnp.float32),
            pltpu.VMEM((G, 1), jnp.float32),
        ],
    )(q0, q1, r2, batch2, w_rel, wcls, bcls)


def kernel(x, edge_index, batch, shape_emb, color_emb, W_node, b_node,
           W1_rel, W1_root, b1, W2_rel, W2_root, b2, W_cls, b_cls):
    xi = x.astype(jnp.int32)
    batch2 = batch.astype(jnp.int32).reshape(NBLK, 1, BLK)
    src = edge_index[0].astype(jnp.int32)
    dst = edge_index[1].astype(jnp.int32)
    grp = NC * NS
    epg = E // grp
    ppg = E_PAD // grp - epg
    padsrc = jnp.zeros((grp, ppg), jnp.int32)
    paddst = jnp.broadcast_to(
        N + (jnp.arange(ppg, dtype=jnp.int32) % NDUMP), (grp, ppg))
    srcp = jnp.concatenate([src.reshape(grp, epg), padsrc], axis=1).reshape(-1)
    dstp = jnp.concatenate([dst.reshape(grp, epg), paddst], axis=1).reshape(-1)
    bn = b_node.reshape(1, -1)
    b1r = b1.reshape(1, -1)
    b2r = b2.reshape(1, -1)
    bc = b_cls.reshape(1, -1)

    h0 = _tc_embed(xi, shape_emb, color_emb, W_node, bn)
    p0, p1 = _build_sc_scatter(200, 3200)(h0, h0, srcp, dstp)
    h1lo, h1hi = _tc_layer1(p0, p1, h0, W1_rel, W1_root, b1r)
    r2 = _tc_root(h1lo, h1hi, W2_root, b2r)
    q0, q1 = _build_sc_scatter(400, 0)(h1lo, h1hi, srcp, dstp)
    return _tc_layer2_pool(q0, q1, r2, batch2, W2_rel, W_cls, bc)
